# baseline (speedup 1.0000x reference)
import functools

import jax
import jax.numpy as jnp
from jax import lax
from jax.experimental import pallas as pl
from jax.experimental.pallas import tpu as pltpu
from jax.experimental.pallas import tpu_sc as plsc

K = 4
R = 2048
NC = 2
NS = 16
NW = NC * NS
RS = 928
ROWS_PER_W = RS // NW
BLK = 112


def _make_sc_upsample():
    mesh = plsc.VectorSubcoreMesh(core_axis_name="c", subcore_axis_name="s")

    @functools.partial(
        pl.kernel,
        mesh=mesh,
        out_type=jax.ShapeDtypeStruct((RS, K, 4, 1024), jnp.float32),
        scratch_types=[
            pltpu.VMEM((16, 1, 4, 1024), jnp.float32),
            pltpu.SemaphoreType.DMA,
            pltpu.SemaphoreType.DMA,
            pltpu.SemaphoreType.DMA,
            pltpu.SemaphoreType.DMA,
        ],
        compiler_params=pltpu.CompilerParams(use_tc_tiling_on_sc=True),
    )
    def upsample(xs_hbm, out_hbm, buf, lsem0, lsem1, ssem0, ssem1):
        wid = lax.axis_index("s") * NC + lax.axis_index("c")
        base = wid * ROWS_PER_W
        lsems = (lsem0, lsem1)
        ssems = (ssem0, ssem1)
        NBUF = 8
        G = (ROWS_PER_W + NBUF - 1) // NBUF

        def batch_rows(g):
            return NBUF if (g + 1) * NBUF <= ROWS_PER_W else ROWS_PER_W - g * NBUF

        loads = [None] * G
        stores = [[] for _ in range(G)]

        def issue_load(g):
            par = g % 2
            return pltpu.async_copy(
                xs_hbm.at[pl.ds(base + g * NBUF, batch_rows(g))],
                buf.at[pl.ds(par * NBUF, batch_rows(g))],
                lsems[par],
            )

        loads[0] = issue_load(0)
        for g in range(G):
            par = g % 2
            if g + 1 < G:
                if g - 1 >= 0:
                    for st in stores[g - 1]:
                        st.wait()
                loads[g + 1] = issue_load(g + 1)
            loads[g].wait()
            row0 = base + g * NBUF
            for b in range(batch_rows(g)):
                for r in range(K):
                    stores[g].append(
                        pltpu.async_copy(
                            buf.at[pl.ds(par * NBUF + b, 1)],
                            out_hbm.at[pl.ds(row0 + b, 1), pl.ds(r, 1)],
                            ssems[par],
                        )
                    )
        for g in (G - 2, G - 1):
            for st in stores[g]:
                st.wait()

    return upsample


_sc_upsample = _make_sc_upsample()


def _tc_body(x_ref, o_ref):
    o_ref[...] = jnp.broadcast_to(
        x_ref[...][:, None, :, :], (BLK, K, 4, 1024)
    )


def _tc_upsample(xs):
    n = xs.shape[0]
    return pl.pallas_call(
        _tc_body,
        grid=(n // BLK,),
        in_specs=[pl.BlockSpec((BLK, 4, 1024), lambda i: (i, 0, 0))],
        out_specs=pl.BlockSpec((BLK, K, 4, 1024), lambda i: (i, 0, 0, 0)),
        out_shape=jax.ShapeDtypeStruct((n, K, 4, 1024), jnp.float32),
    )(xs)


def kernel(x, x_short):
    sc_part = _sc_upsample(x_short[:RS].reshape(RS, 1, 4, 1024))
    tc_part = _tc_upsample(x_short[RS:])
    out = jnp.concatenate(
        [sc_part.reshape(RS * K, 4, 1024), tc_part.reshape((R - RS) * K, 4, 1024)],
        axis=0,
    )
    return out


# speedup vs baseline: 2.9753x; 2.9753x over previous
import functools

import jax
import jax.numpy as jnp
from jax import lax
from jax.experimental import pallas as pl
from jax.experimental.pallas import tpu as pltpu
from jax.experimental.pallas import tpu_sc as plsc

K = 4
R = 2048
NC = 2
NS = 16
NW = NC * NS
ROWS_PER_W = R // NW
NBUF = 8
HALF = ROWS_PER_W // 2
G = HALF // NBUF


def _make_sc_upsample():
    mesh = plsc.VectorSubcoreMesh(core_axis_name="c", subcore_axis_name="s")

    @functools.partial(
        pl.kernel,
        mesh=mesh,
        out_type=jax.ShapeDtypeStruct((R, K, 4, 1024), jnp.float32),
        scratch_types=[
            pltpu.VMEM((2 * NBUF, 1, 4, 1024), jnp.float32),
            pltpu.VMEM_SHARED((NS * 2 * NBUF, 1, 4, 1024), jnp.float32),
            pltpu.SemaphoreType.DMA,
            pltpu.SemaphoreType.DMA,
            pltpu.SemaphoreType.DMA,
            pltpu.SemaphoreType.DMA,
            pltpu.SemaphoreType.DMA,
            pltpu.SemaphoreType.DMA,
            pltpu.SemaphoreType.DMA,
            pltpu.SemaphoreType.DMA,
        ],
        compiler_params=pltpu.CompilerParams(use_tc_tiling_on_sc=True),
    )
    def upsample(xs_hbm, out_hbm, tbuf, sbuf, la0, la1, sa0, sa1,
                 lb0, lb1, sb0, sb1):
        c = lax.axis_index("c")
        s = lax.axis_index("s")
        wid = s * NC + c
        base_a = wid * ROWS_PER_W
        base_b = base_a + HALF
        soff = s * 2 * NBUF
        lsems = ((la0, la1), (lb0, lb1))
        ssems = ((sa0, sa1), (sb0, sb1))
        bufs = (tbuf, sbuf)
        offs = (0, soff)
        bases = (base_a, base_b)

        loads = [[None] * G, [None] * G]
        stores = [[[] for _ in range(G)], [[] for _ in range(G)]]

        def issue_load(p, g):
            par = g % 2
            return pltpu.async_copy(
                xs_hbm.at[pl.ds(bases[p] + g * NBUF, NBUF)],
                bufs[p].at[pl.ds(offs[p] + par * NBUF, NBUF)],
                lsems[p][par],
            )

        loads[0][0] = issue_load(0, 0)
        loads[1][0] = issue_load(1, 0)
        for g in range(G):
            par = g % 2
            if g + 1 < G:
                if g - 1 >= 0:
                    for p in (0, 1):
                        for st in stores[p][g - 1]:
                            st.wait()
                loads[0][g + 1] = issue_load(0, g + 1)
                loads[1][g + 1] = issue_load(1, g + 1)
            loads[0][g].wait()
            loads[1][g].wait()
            for p in (0, 1):
                row0 = bases[p] + g * NBUF
                for b in range(NBUF):
                    for r in range(K):
                        stores[p][g].append(
                            pltpu.async_copy(
                                bufs[p].at[pl.ds(offs[p] + par * NBUF + b, 1)],
                                out_hbm.at[pl.ds(row0 + b, 1), pl.ds(r, 1)],
                                ssems[p][par],
                            )
                        )
        for p in (0, 1):
            for g in (G - 2, G - 1):
                for st in stores[p][g]:
                    st.wait()

    return upsample


_sc_upsample = _make_sc_upsample()


def kernel(x, x_short):
    xs = x_short.reshape(R, 1, 4, 1024)
    out = _sc_upsample(xs)
    return out.reshape(R * K, 4, 1024)


# speedup vs baseline: 3.0616x; 1.0290x over previous
import functools

import jax
import jax.numpy as jnp
from jax import lax
from jax.experimental import pallas as pl
from jax.experimental.pallas import tpu as pltpu
from jax.experimental.pallas import tpu_sc as plsc

K = 4
R = 2048
NC = 2
NS = 16
NW = NC * NS
ROWS_PER_W = R // NW
NBUF = 8
G = ROWS_PER_W // NBUF


def _make_sc_upsample():
    mesh = plsc.VectorSubcoreMesh(core_axis_name="c", subcore_axis_name="s")

    @functools.partial(
        pl.kernel,
        mesh=mesh,
        out_type=jax.ShapeDtypeStruct((R, K, 4, 1024), jnp.float32),
        scratch_types=[
            pltpu.VMEM((2 * NBUF, 1, 4, 1024), jnp.float32),
            pltpu.SemaphoreType.DMA,
            pltpu.SemaphoreType.DMA,
            pltpu.SemaphoreType.DMA,
            pltpu.SemaphoreType.DMA,
        ],
        compiler_params=pltpu.CompilerParams(use_tc_tiling_on_sc=True),
    )
    def upsample(xs_hbm, out_hbm, buf, lsem0, lsem1, ssem0, ssem1):
        wid = lax.axis_index("s") * NC + lax.axis_index("c")
        base = wid * ROWS_PER_W
        lsems = (lsem0, lsem1)
        ssems = (ssem0, ssem1)

        loads = [None] * G
        stores = [[] for _ in range(G)]

        def issue_load(g):
            par = g % 2
            return pltpu.async_copy(
                xs_hbm.at[pl.ds(base + g * NBUF, NBUF)],
                buf.at[pl.ds(par * NBUF, NBUF)],
                lsems[par],
            )

        loads[0] = issue_load(0)
        for g in range(G):
            par = g % 2
            if g + 1 < G:
                if g - 1 >= 0:
                    for st in stores[g - 1]:
                        st.wait()
                loads[g + 1] = issue_load(g + 1)
            loads[g].wait()
            row0 = base + g * NBUF
            for b in range(NBUF):
                for r in range(K):
                    stores[g].append(
                        pltpu.async_copy(
                            buf.at[pl.ds(par * NBUF + b, 1)],
                            out_hbm.at[pl.ds(row0 + b, 1), pl.ds(r, 1)],
                            ssems[par],
                        )
                    )
        for g in (G - 2, G - 1):
            for st in stores[g]:
                st.wait()

    return upsample


_sc_upsample = _make_sc_upsample()


def kernel(x, x_short):
    xs = x_short.reshape(R, 1, 4, 1024)
    out = _sc_upsample(xs)
    return out.reshape(R * K, 4, 1024)
